# baseline (speedup 1.0000x reference)
import functools
import math

import jax
import jax.numpy as jnp
from jax import lax
from jax.experimental import pallas as pl
from jax.experimental.pallas import tpu as pltpu
from jax.experimental.pallas import tpu_sc as plsc

D_MODEL = 64
SCALE = math.sqrt(D_MODEL)


@functools.partial(jax.jit, static_argnames=("B", "D"))
def _emb_lookup(idx_flat, table_packed, *, B, D):
    info = plsc.get_sparse_core_info()
    NC, NS, L = info.num_cores, info.num_subcores, info.num_lanes
    NW = NC * NS
    assert B % NW == 0
    b_per_w = B // NW
    C = 400
    n_chunks = b_per_w // C
    assert D % L == 0

    mesh = plsc.VectorSubcoreMesh(core_axis_name="c", subcore_axis_name="s")

    @functools.partial(
        pl.kernel,
        mesh=mesh,
        out_type=jax.ShapeDtypeStruct((B, D), jnp.float32),
        scratch_types=[
            pltpu.VMEM((b_per_w,), jnp.int32),
            pltpu.VMEM((C, 2 * D), jnp.float32),
            pltpu.VMEM((C, D), jnp.float32),
            pltpu.SemaphoreType.DMA,
        ],
    )
    def k(idx_hbm, tp_hbm, out_hbm, idx_v, rows_v, sel_v, sem):
        wid = lax.axis_index("s") * NC + lax.axis_index("c")
        base = wid * b_per_w
        pltpu.sync_copy(idx_hbm.at[pl.ds(base, b_per_w)], idx_v)

        def chunk(j, carry0):
            def issue(r, carry):
                vec = idx_v[pl.ds(j * C + r * L, L)]
                pair = lax.shift_right_logical(vec, 1)
                for t in range(L):
                    pltpu.make_async_copy(
                        tp_hbm.at[pair[t]], rows_v.at[r * L + t], sem
                    ).start()
                return carry

            lax.fori_loop(0, C // L, issue, 0)
            pltpu.make_async_copy(
                tp_hbm.at[pl.ds(0, C)], rows_v, sem
            ).wait()

            def sel_scale(q, carry):
                vec = idx_v[pl.ds(j * C + q * L, L)]
                off = lax.shift_left(
                    lax.bitwise_and(vec, jnp.int32(1)), 6
                )
                for t in range(L):
                    r = q * L + t
                    for g in range(D // L):
                        sel_v[r, pl.ds(g * L, L)] = (
                            rows_v[r, pl.ds(off[t] + g * L, L)] * SCALE
                        )
                return carry

            lax.fori_loop(0, C // L, sel_scale, 0)

            pltpu.sync_copy(sel_v, out_hbm.at[pl.ds(base + j * C, C)])
            return carry0

        lax.fori_loop(0, n_chunks, chunk, 0)

    return k(idx_flat, table_packed)


def kernel(x, word_emb_weight):
    B = x.shape[0] * x.shape[1]
    D = word_emb_weight.shape[1]
    idx_flat = x.reshape(B)
    table_packed = word_emb_weight.reshape(-1, 2 * D)
    out = _emb_lookup(idx_flat, table_packed, B=B, D=D)
    return out.reshape(x.shape[0], x.shape[1], D)


# speedup vs baseline: 1.9817x; 1.9817x over previous
import functools
import math

import jax
import jax.numpy as jnp
from jax import lax
from jax.experimental import pallas as pl
from jax.experimental.pallas import tpu as pltpu
from jax.experimental.pallas import tpu_sc as plsc

D_MODEL = 64
SCALE = math.sqrt(D_MODEL)


@functools.partial(jax.jit, static_argnames=("B", "D", "H"))
def _emb_lookup(idx_flat, table3, *, B, D, H):
    info = plsc.get_sparse_core_info()
    NC, NS, L = info.num_cores, info.num_subcores, info.num_lanes
    NW = NC * NS
    assert B % NW == 0
    b_per_w = B // NW
    C = 800
    n_chunks = b_per_w // C
    assert D % L == 0

    mesh = plsc.VectorSubcoreMesh(core_axis_name="c", subcore_axis_name="s")

    @functools.partial(
        pl.kernel,
        mesh=mesh,
        out_type=jax.ShapeDtypeStruct((B, D), jnp.float32),
        scratch_types=[
            pltpu.VMEM((b_per_w,), jnp.int32),
            pltpu.VMEM((C, D), jnp.float32),
            pltpu.SemaphoreType.DMA,
        ],
    )
    def k(idx_hbm, tp_hbm, out_hbm, idx_v, rows_v, sem):
        wid = lax.axis_index("s") * NC + lax.axis_index("c")
        base = wid * b_per_w
        pltpu.sync_copy(idx_hbm.at[pl.ds(base, b_per_w)], idx_v)

        def chunk(j, carry0):
            def issue(r, carry):
                vec = idx_v[pl.ds(j * C + r * L, L)]
                hi = jnp.where(vec >= H, jnp.int32(1), jnp.int32(0))
                lo = vec - hi * H
                for t in range(L):
                    pltpu.make_async_copy(
                        tp_hbm.at[hi[t], lo[t]], rows_v.at[r * L + t], sem
                    ).start()
                return carry

            lax.fori_loop(0, C // L, issue, 0)
            pltpu.make_async_copy(
                tp_hbm.at[0, pl.ds(0, C)], rows_v, sem
            ).wait()

            def scale_row(i, carry):
                for g in range(D // L):
                    sl = (i, pl.ds(g * L, L))
                    rows_v[sl] = rows_v[sl] * SCALE
                return carry

            lax.fori_loop(0, C, scale_row, 0)

            pltpu.sync_copy(rows_v, out_hbm.at[pl.ds(base + j * C, C)])
            return carry0

        lax.fori_loop(0, n_chunks, chunk, 0)

    return k(idx_flat, table3)


def kernel(x, word_emb_weight):
    B = x.shape[0] * x.shape[1]
    V, D = word_emb_weight.shape
    idx_flat = x.reshape(B)
    table3 = word_emb_weight.reshape(2, V // 2, D)
    out = _emb_lookup(idx_flat, table3, B=B, D=D, H=V // 2)
    return out.reshape(x.shape[0], x.shape[1], D)
